# baseline (speedup 1.0000x reference)
import functools

import jax
import jax.numpy as jnp
from jax import lax
from jax.experimental import pallas as pl
from jax.experimental.pallas import tpu as pltpu
from jax.experimental.pallas import tpu_sc as plsc

V = 1000000
D = 64
B = 4096
S = 200
LANES = 16
NW = 32
NBUF = 4
CB = 128

NBUF1 = 2
CB1 = 256
N_FULL_BLOCKS = V // CB1
TAIL_W = V - N_FULL_BLOCKS * CB1
G_MAIN = N_FULL_BLOCKS // NW
N_EXTRA = N_FULL_BLOCKS - G_MAIN * NW


def _wid():
    return lax.axis_index("s") * 2 + lax.axis_index("c")


@functools.cache
def _build_detile():
    mesh = plsc.VectorSubcoreMesh(core_axis_name="c", subcore_axis_name="s")

    @functools.partial(
        pl.kernel,
        mesh=mesh,
        compiler_params=pltpu.CompilerParams(
            use_tc_tiling_on_sc=True, needs_layout_passes=False),
        out_type=jax.ShapeDtypeStruct((V * D,), jnp.float32),
        scratch_types=(
            [pltpu.VMEM((D, CB1), jnp.float32)] * NBUF1
            + [pltpu.VMEM((CB1 * D,), jnp.float32)] * NBUF1
            + [pltpu.VMEM((D, TAIL_W), jnp.float32)]
            + [pltpu.SemaphoreType.DMA] * (2 * NBUF1)
        ),
    )
    def detile(wt_hbm, out_hbm, *refs):
        bufs = refs[0:NBUF1]
        tbs = refs[NBUF1:2 * NBUF1]
        buf2 = refs[2 * NBUF1]
        sis = refs[2 * NBUF1 + 1:2 * NBUF1 + 1 + NBUF1]
        sos = refs[2 * NBUF1 + 1 + NBUF1:]
        w = _wid()
        lanes64 = jnp.arange(LANES, dtype=jnp.int32) * D

        def c0_of(g):
            return (w + NW * g) * CB1

        def transpose_block(src, dst):
            def lrow(l0, carry):
                base = l0 * LANES * D

                @plsc.parallel_loop(0, D, unroll=8)
                def dloop(d):
                    vals = src[d, pl.ds(l0 * LANES, LANES)]
                    plsc.store_scatter(dst, [lanes64 + (base + d)], vals)

                return carry

            lax.fori_loop(0, CB1 // LANES, lrow, 0)

        for b in range(NBUF1):
            pltpu.async_copy(
                wt_hbm.at[:, pl.ds(c0_of(b), CB1)], bufs[b], sis[b])

        def step(j, carry):
            for b in range(NBUF1):
                g = NBUF1 * j + b
                c0 = c0_of(g)
                pltpu.make_async_copy(
                    wt_hbm.at[:, pl.ds(c0, CB1)], bufs[b], sis[b]).wait()

                @pl.when(j >= 1)
                def _():
                    pltpu.make_async_copy(
                        tbs[b], out_hbm.at[pl.ds(0, CB1 * D)], sos[b]).wait()

                transpose_block(bufs[b], tbs[b])
                pltpu.async_copy(
                    tbs[b], out_hbm.at[pl.ds(c0 * D, CB1 * D)], sos[b])

                @pl.when(g + NBUF1 < G_MAIN)
                def _():
                    pltpu.async_copy(
                        wt_hbm.at[:, pl.ds(c0_of(g + NBUF1), CB1)],
                        bufs[b], sis[b])

            return carry

        lax.fori_loop(0, G_MAIN // NBUF1, step, 0)
        for b in range(NBUF1):
            pltpu.make_async_copy(
                tbs[b], out_hbm.at[pl.ds(0, CB1 * D)], sos[b]).wait()

        @pl.when(w < N_EXTRA)
        def _extra():
            cb = G_MAIN * NW + w
            c0 = cb * CB1
            pltpu.sync_copy(wt_hbm.at[:, pl.ds(c0, CB1)], bufs[0])
            transpose_block(bufs[0], tbs[0])
            pltpu.sync_copy(tbs[0], out_hbm.at[pl.ds(c0 * D, CB1 * D)])

        @pl.when(w == NW - 1)
        def _tail():
            c0 = N_FULL_BLOCKS * CB1
            pltpu.sync_copy(wt_hbm.at[:, pl.ds(c0, TAIL_W)], buf2)

            def lrow(l0, carry):
                base = l0 * LANES * D

                @plsc.parallel_loop(0, D, unroll=8)
                def dloop(d):
                    vals = buf2[d, pl.ds(l0 * LANES, LANES)]
                    plsc.store_scatter(tbs[0], [lanes64 + (base + d)], vals)

                return carry

            lax.fori_loop(0, TAIL_W // LANES, lrow, 0)
            pltpu.sync_copy(tbs[0].at[pl.ds(0, TAIL_W * D)],
                            out_hbm.at[pl.ds(c0 * D, TAIL_W * D)])

    return detile


@functools.cache
def _build_gather():
    mesh = plsc.VectorSubcoreMesh(core_axis_name="c", subcore_axis_name="s")
    n_c = B // CB

    @functools.partial(
        pl.kernel,
        mesh=mesh,
        compiler_params=pltpu.CompilerParams(
            use_tc_tiling_on_sc=False, needs_layout_passes=False),
        out_type=jax.ShapeDtypeStruct((S, D // 8, n_c, 8, CB), jnp.float32),
        scratch_types=(
            [pltpu.VMEM((S, CB), jnp.int32)]
            + [pltpu.VMEM((CB, D), jnp.float32)] * NBUF
            + [pltpu.VMEM((D // 8, 8, CB), jnp.float32)] * NBUF
            + [pltpu.SemaphoreType.DMA] * (2 * NBUF)
        ),
    )
    def gather(table_hbm, idxt_hbm, out_hbm, idx_all, *refs):
        rows = refs[0:NBUF]
        ovs = refs[NBUF:2 * NBUF]
        sgs = refs[2 * NBUF:3 * NBUF]
        sos = refs[3 * NBUF:]
        c = _wid()
        lanes = jnp.arange(LANES, dtype=jnp.int32)

        pltpu.sync_copy(idxt_hbm.at[:, pl.ds(c * CB, CB)], idx_all)
        for b in range(NBUF):
            pltpu.async_copy(table_hbm.at[idx_all.at[b]], rows[b], sgs[b])

        def transpose_unit(src, dst):
            def lrow(l0, carry):
                rowidx = lanes + l0 * LANES

                @plsc.parallel_loop(0, D, unroll=8)
                def dloop(d):
                    vals = plsc.load_gather(
                        src, [rowidx, jnp.full((LANES,), d, jnp.int32)])
                    dst[d // 8, d % 8, pl.ds(l0 * LANES, LANES)] = vals

                return carry

            lax.fori_loop(0, CB // LANES, lrow, 0)

        def step(j, carry):
            for b in range(NBUF):
                s = NBUF * j + b
                pltpu.make_async_copy(
                    table_hbm.at[idx_all.at[s]], rows[b], sgs[b]).wait()

                @pl.when(j >= 1)
                def _():
                    pltpu.make_async_copy(
                        ovs[b], out_hbm.at[0, :, c], sos[b]).wait()

                transpose_unit(rows[b], ovs[b])
                pltpu.async_copy(ovs[b], out_hbm.at[s, :, c], sos[b])

                @pl.when(s + NBUF < S)
                def _():
                    pltpu.async_copy(
                        table_hbm.at[idx_all.at[s + NBUF]], rows[b], sgs[b])

            return carry

        lax.fori_loop(0, S // NBUF, step, 0)
        for b in range(NBUF):
            pltpu.make_async_copy(
                ovs[b], out_hbm.at[0, :, c], sos[b]).wait()

    return gather


def kernel(indices, weight):
    table = _build_detile()(weight.T).reshape(V, D)
    o5 = _build_gather()(table, indices.T)
    return o5.transpose(2, 4, 0, 1, 3).reshape(B, S, D)


# speedup vs baseline: 3.9662x; 3.9662x over previous
import functools

import jax
import jax.numpy as jnp
from jax import lax
from jax.experimental import pallas as pl
from jax.experimental.pallas import tpu as pltpu
from jax.experimental.pallas import tpu_sc as plsc

V = 1000000
D = 64
B = 4096
S = 200
LANES = 16
NW = 32
NBUF = 4

CB = 128
N_FULL_BLOCKS = V // CB
TAIL_W = V - N_FULL_BLOCKS * CB
G_MAIN = N_FULL_BLOCKS // NW
N_EXTRA = N_FULL_BLOCKS - G_MAIN * NW


def _wid():
    return lax.axis_index("s") * 2 + lax.axis_index("c")


@functools.cache
def _build_detile():
    mesh = plsc.VectorSubcoreMesh(core_axis_name="c", subcore_axis_name="s")

    @functools.partial(
        pl.kernel,
        mesh=mesh,
        compiler_params=pltpu.CompilerParams(
            use_tc_tiling_on_sc=True, needs_layout_passes=False),
        out_type=jax.ShapeDtypeStruct((V * D,), jnp.float32),
        scratch_types=(
            [pltpu.VMEM((D, CB), jnp.float32)] * NBUF
            + [pltpu.VMEM((CB * D,), jnp.float32)] * NBUF
            + [pltpu.VMEM((D, TAIL_W), jnp.float32)]
            + [pltpu.SemaphoreType.DMA] * (2 * NBUF)
        ),
    )
    def detile(wt_hbm, out_hbm, *refs):
        bufs = refs[0:NBUF]
        tbs = refs[NBUF:2 * NBUF]
        buf2 = refs[2 * NBUF]
        sis = refs[2 * NBUF + 1:2 * NBUF + 1 + NBUF]
        sos = refs[2 * NBUF + 1 + NBUF:]
        w = _wid()
        lanes = jnp.arange(LANES, dtype=jnp.int32)

        def c0_of(g):
            return (w + NW * g) * CB

        def transpose_block(src, dst, width):
            def lrow(l0, carry):
                lvec = l0 * LANES + lanes
                lvec_d = lvec * D
                for d0 in range(0, D, LANES):

                    @plsc.parallel_loop(0, LANES, unroll=4)
                    def jloop(j):
                        dvec = d0 + ((lanes + j) & (LANES - 1))
                        vals = plsc.load_gather(src, [dvec, lvec])
                        plsc.store_scatter(dst, [lvec_d + dvec], vals)

                return carry

            lax.fori_loop(0, width // LANES, lrow, 0)

        for b in range(NBUF):
            pltpu.async_copy(
                wt_hbm.at[:, pl.ds(c0_of(b), CB)], bufs[b], sis[b])

        def step(j, carry):
            for b in range(NBUF):
                g = NBUF * j + b
                c0 = c0_of(g)
                pltpu.make_async_copy(
                    wt_hbm.at[:, pl.ds(c0, CB)], bufs[b], sis[b]).wait()

                @pl.when(j >= 1)
                def _():
                    pltpu.make_async_copy(
                        tbs[b], out_hbm.at[pl.ds(0, CB * D)], sos[b]).wait()

                transpose_block(bufs[b], tbs[b], CB)
                pltpu.async_copy(
                    tbs[b], out_hbm.at[pl.ds(c0 * D, CB * D)], sos[b])

                @pl.when(g + NBUF < G_MAIN)
                def _():
                    pltpu.async_copy(
                        wt_hbm.at[:, pl.ds(c0_of(g + NBUF), CB)],
                        bufs[b], sis[b])

            return carry

        lax.fori_loop(0, G_MAIN // NBUF, step, 0)
        for b in range(NBUF):
            pltpu.make_async_copy(
                tbs[b], out_hbm.at[pl.ds(0, CB * D)], sos[b]).wait()

        @pl.when(w < N_EXTRA)
        def _extra():
            cb = G_MAIN * NW + w
            c0 = cb * CB
            pltpu.sync_copy(wt_hbm.at[:, pl.ds(c0, CB)], bufs[0])
            transpose_block(bufs[0], tbs[0], CB)
            pltpu.sync_copy(tbs[0], out_hbm.at[pl.ds(c0 * D, CB * D)])

        @pl.when(w == NW - 1)
        def _tail():
            c0 = N_FULL_BLOCKS * CB
            pltpu.sync_copy(wt_hbm.at[:, pl.ds(c0, TAIL_W)], buf2)
            transpose_block(buf2, tbs[0], TAIL_W)
            pltpu.sync_copy(tbs[0].at[pl.ds(0, TAIL_W * D)],
                            out_hbm.at[pl.ds(c0 * D, TAIL_W * D)])

    return detile


@functools.cache
def _build_gather():
    mesh = plsc.VectorSubcoreMesh(core_axis_name="c", subcore_axis_name="s")
    n_c = B // CB

    @functools.partial(
        pl.kernel,
        mesh=mesh,
        compiler_params=pltpu.CompilerParams(
            use_tc_tiling_on_sc=False, needs_layout_passes=False),
        out_type=jax.ShapeDtypeStruct((S, D // 8, n_c, 8, CB), jnp.float32),
        scratch_types=(
            [pltpu.VMEM((S, CB), jnp.int32)]
            + [pltpu.VMEM((CB, D), jnp.float32)] * NBUF
            + [pltpu.VMEM((D // 8, 8, CB), jnp.float32)] * NBUF
            + [pltpu.SemaphoreType.DMA] * (2 * NBUF)
        ),
    )
    def gather(table_hbm, idxt_hbm, out_hbm, idx_all, *refs):
        rows = refs[0:NBUF]
        ovs = refs[NBUF:2 * NBUF]
        sgs = refs[2 * NBUF:3 * NBUF]
        sos = refs[3 * NBUF:]
        c = _wid()
        lanes = jnp.arange(LANES, dtype=jnp.int32)

        pltpu.sync_copy(idxt_hbm.at[:, pl.ds(c * CB, CB)], idx_all)
        for b in range(NBUF):
            pltpu.async_copy(table_hbm.at[idx_all.at[b]], rows[b], sgs[b])

        def transpose_unit(src, dst):
            def lrow(l0, carry):
                lvec = l0 * LANES + lanes
                for d0 in range(0, D, LANES):

                    @plsc.parallel_loop(0, LANES, unroll=4)
                    def jloop(j):
                        dvec = d0 + ((lanes + j) & (LANES - 1))
                        vals = plsc.load_gather(src, [lvec, dvec])
                        plsc.store_scatter(
                            dst, [dvec // 8, dvec % 8, lvec], vals)

                return carry

            lax.fori_loop(0, CB // LANES, lrow, 0)

        def step(j, carry):
            for b in range(NBUF):
                s = NBUF * j + b
                pltpu.make_async_copy(
                    table_hbm.at[idx_all.at[s]], rows[b], sgs[b]).wait()

                @pl.when(j >= 1)
                def _():
                    pltpu.make_async_copy(
                        ovs[b], out_hbm.at[0, :, c], sos[b]).wait()

                transpose_unit(rows[b], ovs[b])
                pltpu.async_copy(ovs[b], out_hbm.at[s, :, c], sos[b])

                @pl.when(s + NBUF < S)
                def _():
                    pltpu.async_copy(
                        table_hbm.at[idx_all.at[s + NBUF]], rows[b], sgs[b])

            return carry

        lax.fori_loop(0, S // NBUF, step, 0)
        for b in range(NBUF):
            pltpu.make_async_copy(
                ovs[b], out_hbm.at[0, :, c], sos[b]).wait()

    return gather


def kernel(indices, weight):
    table = _build_detile()(weight.T).reshape(V, D)
    o5 = _build_gather()(table, indices.T)
    return o5.transpose(2, 4, 0, 1, 3).reshape(B, S, D)
